# baseline (speedup 1.0000x reference)
import functools

import jax
import jax.numpy as jnp
from jax import lax
from jax.experimental import pallas as pl
from jax.experimental.pallas import tpu as pltpu
from jax.experimental.pallas import tpu_sc as plsc

N = 16384
C = 128
K = 48
FF = 256
H = 8
DH = C // H

SC_CORES = 2
SC_SUBCORES = 16
NW = SC_CORES * SC_SUBCORES
GCH = 128


def _pre_body(x_ref, idx_ref, ki_ref, wk3_ref, bk3_ref, wq3_ref, bq3_ref,
              wk_ref, bk_ref, wv_ref, bv_ref, wq_ref, bq_ref,
              kv_ref, q_ref, sidx_ref):
    idxf = idx_ref[...].astype(jnp.float32)
    cx = (idxf[:, 3:4] + 0.5) * 0.05
    cy = (idxf[:, 2:3] + 0.5) * 0.05 - 40.0
    cz = (idxf[:, 1:2] + 0.5) * 0.1 - 3.0
    cb = jnp.concatenate([cx, cy, cz], axis=1)
    x = x_ref[...]
    pos_k = jnp.maximum(
        jnp.dot(cb, wk3_ref[...], preferred_element_type=jnp.float32)
        + bk3_ref[...], 0.0)
    pos_q = jnp.maximum(
        jnp.dot(cb, wq3_ref[...], preferred_element_type=jnp.float32)
        + bq3_ref[...], 0.0)
    keysrc = x + pos_k
    kv_ref[:, :C] = jnp.dot(keysrc, wk_ref[...],
                            preferred_element_type=jnp.float32) + bk_ref[...]
    kv_ref[:, C:] = jnp.dot(keysrc, wv_ref[...],
                            preferred_element_type=jnp.float32) + bv_ref[...]
    q_ref[...] = (jnp.dot(x + pos_q, wq_ref[...],
                          preferred_element_type=jnp.float32)
                  + bq_ref[...]) * (1.0 / (DH ** 0.5))
    sidx_ref[...] = jnp.maximum(ki_ref[...], 0)


def _sc_gather_body(kv_hbm, sidx_hbm, out_hbm, idx_v, rows_v, sem):
    wid = lax.axis_index("s") * SC_CORES + lax.axis_index("c")
    per_w = (N * K) // NW
    base = wid * per_w

    def body(i, carry):
        off = base + i * GCH
        pltpu.sync_copy(sidx_hbm.at[pl.ds(off, GCH)], idx_v)
        pltpu.async_copy(kv_hbm.at[idx_v], rows_v, sem).wait()
        pltpu.sync_copy(rows_v, out_hbm.at[pl.ds(off, GCH)])
        return carry

    lax.fori_loop(0, per_w // GCH, body, 0)


@functools.cache
def _make_sc_gather():
    return pl.kernel(
        _sc_gather_body,
        out_type=jax.ShapeDtypeStruct((N * K, 2 * C), jnp.float32),
        mesh=plsc.VectorSubcoreMesh(core_axis_name="c", subcore_axis_name="s",
                                    num_cores=SC_CORES,
                                    num_subcores=SC_SUBCORES),
        scratch_types=[
            pltpu.VMEM((GCH,), jnp.int32),
            pltpu.VMEM((GCH, 2 * C), jnp.float32),
            pltpu.SemaphoreType.DMA,
        ],
    )


def _sc_gather(kv_tab, sidx):
    return _make_sc_gather()(kv_tab, sidx)


def _attn_body(kv_ref, q_ref, x_ref, ki_ref, wo_ref, bo_ref, s1_ref, st_ref):
    b = q_ref.shape[0]
    kv = kv_ref[...]
    q = q_ref[...]
    kk = kv[:, :, :C]
    vv = kv[:, :, C:]
    prod = kk * q[:, None, :]
    mask = ki_ref[...] < 0
    attn_cols = []
    for h in range(H):
        lh = jnp.sum(prod[:, :, h * DH:(h + 1) * DH], axis=2)
        lh = jnp.where(mask, -1e9, lh)
        m = jnp.max(lh, axis=1, keepdims=True)
        e = jnp.exp(lh - m)
        s = jnp.sum(e, axis=1, keepdims=True)
        a = e / s
        attn_cols.append(jnp.broadcast_to(a[:, :, None], (b, K, DH)))
    attn_full = jnp.concatenate(attn_cols, axis=2)
    ctx = jnp.sum(attn_full * vv, axis=1)
    attend = jnp.dot(ctx, wo_ref[...],
                     preferred_element_type=jnp.float32) + bo_ref[...]
    s1 = x_ref[...] + attend
    s1_ref[...] = s1
    blk = jnp.concatenate(
        [jnp.sum(s1, axis=0, keepdims=True),
         jnp.sum(s1 * s1, axis=0, keepdims=True),
         jnp.zeros((6, C), jnp.float32)], axis=0)

    @pl.when(pl.program_id(0) == 0)
    def _():
        st_ref[...] = jnp.zeros_like(st_ref)

    st_ref[...] += blk


def _ffn_body(s1_ref, st1_ref, g1_ref, b1_ref, w1_ref, bb1_ref,
              w2_ref, bb2_ref, s2_ref, st_ref):
    mean = st1_ref[0:1, :] * (1.0 / N)
    var = st1_ref[1:2, :] * (1.0 / N) - mean * mean
    vf = ((s1_ref[...] - mean) * lax.rsqrt(var + 1e-5) * g1_ref[...]
          + b1_ref[...])
    a = jnp.maximum(jnp.dot(vf, w1_ref[...],
                            preferred_element_type=jnp.float32)
                    + bb1_ref[...], 0.0)
    act = jnp.dot(a, w2_ref[...],
                  preferred_element_type=jnp.float32) + bb2_ref[...]
    s2 = vf + act
    s2_ref[...] = s2
    blk = jnp.concatenate(
        [jnp.sum(s2, axis=0, keepdims=True),
         jnp.sum(s2 * s2, axis=0, keepdims=True),
         jnp.zeros((6, C), jnp.float32)], axis=0)

    @pl.when(pl.program_id(0) == 0)
    def _():
        st_ref[...] = jnp.zeros_like(st_ref)

    st_ref[...] += blk


def _proj_body(s2_ref, st2_ref, g2_ref, b2_ref, wo_ref, bo_ref,
               p_ref, st_ref):
    mean = st2_ref[0:1, :] * (1.0 / N)
    var = st2_ref[1:2, :] * (1.0 / N) - mean * mean
    vf = ((s2_ref[...] - mean) * lax.rsqrt(var + 1e-5) * g2_ref[...]
          + b2_ref[...])
    p = jnp.dot(vf, wo_ref[...],
                preferred_element_type=jnp.float32) + bo_ref[...]
    p_ref[...] = p
    blk = jnp.concatenate(
        [jnp.sum(p, axis=0, keepdims=True),
         jnp.sum(p * p, axis=0, keepdims=True),
         jnp.zeros((6, C), jnp.float32)], axis=0)

    @pl.when(pl.program_id(0) == 0)
    def _():
        st_ref[...] = jnp.zeros_like(st_ref)

    st_ref[...] += blk


def _final_body(p_ref, st3_ref, g3_ref, b3_ref, out_ref):
    mean = st3_ref[0:1, :] * (1.0 / N)
    var = st3_ref[1:2, :] * (1.0 / N) - mean * mean
    out_ref[...] = jnp.maximum(
        (p_ref[...] - mean) * lax.rsqrt(var + 1e-5) * g3_ref[...]
        + b3_ref[...], 0.0)


def _row(v):
    return v.reshape(1, -1)


def kernel(x, indices, key_indices, in_proj_w, in_proj_b, out_proj_w,
           out_proj_b, q_pos_w, q_pos_b, k_pos_w, k_pos_b, norm1_g, norm1_b,
           norm2_g, norm2_b, lin1_w, lin1_b, lin2_w, lin2_b, out_w, out_b,
           bn_out_g, bn_out_b):
    f32 = jnp.float32
    wq = in_proj_w[:C].T
    wk = in_proj_w[C:2 * C].T
    wv = in_proj_w[2 * C:].T
    bq, bk, bv = in_proj_b[:C], in_proj_b[C:2 * C], in_proj_b[2 * C:]

    BP = 1024
    full = lambda shape: pl.BlockSpec(shape, lambda i: (0, 0))
    kv_tab, q_tab, sidx = pl.pallas_call(
        _pre_body,
        grid=(N // BP,),
        in_specs=[
            pl.BlockSpec((BP, C), lambda i: (i, 0)),
            pl.BlockSpec((BP, 4), lambda i: (i, 0)),
            pl.BlockSpec((BP, K), lambda i: (i, 0)),
            full((3, C)), full((1, C)), full((3, C)), full((1, C)),
            full((C, C)), full((1, C)), full((C, C)), full((1, C)),
            full((C, C)), full((1, C)),
        ],
        out_specs=[
            pl.BlockSpec((BP, 2 * C), lambda i: (i, 0)),
            pl.BlockSpec((BP, C), lambda i: (i, 0)),
            pl.BlockSpec((BP, K), lambda i: (i, 0)),
        ],
        out_shape=[
            jax.ShapeDtypeStruct((N, 2 * C), f32),
            jax.ShapeDtypeStruct((N, C), f32),
            jax.ShapeDtypeStruct((N, K), jnp.int32),
        ],
    )(x, indices, key_indices, k_pos_w.T, _row(k_pos_b), q_pos_w.T,
      _row(q_pos_b), wk, _row(bk), wv, _row(bv), wq, _row(bq))

    kv_gathered = _sc_gather(kv_tab, sidx.reshape(N * K))
    kv_gathered = kv_gathered.reshape(N, K, 2 * C)

    BA = 64
    s1, st1 = pl.pallas_call(
        _attn_body,
        grid=(N // BA,),
        in_specs=[
            pl.BlockSpec((BA, K, 2 * C), lambda i: (i, 0, 0)),
            pl.BlockSpec((BA, C), lambda i: (i, 0)),
            pl.BlockSpec((BA, C), lambda i: (i, 0)),
            pl.BlockSpec((BA, K), lambda i: (i, 0)),
            full((C, C)), full((1, C)),
        ],
        out_specs=[
            pl.BlockSpec((BA, C), lambda i: (i, 0)),
            pl.BlockSpec((8, C), lambda i: (0, 0)),
        ],
        out_shape=[
            jax.ShapeDtypeStruct((N, C), f32),
            jax.ShapeDtypeStruct((8, C), f32),
        ],
    )(kv_gathered, q_tab, x, key_indices, out_proj_w.T, _row(out_proj_b))

    BF = 2048
    s2, st2 = pl.pallas_call(
        _ffn_body,
        grid=(N // BF,),
        in_specs=[
            pl.BlockSpec((BF, C), lambda i: (i, 0)),
            full((8, C)), full((1, C)), full((1, C)),
            full((C, FF)), full((1, FF)), full((FF, C)), full((1, C)),
        ],
        out_specs=[
            pl.BlockSpec((BF, C), lambda i: (i, 0)),
            pl.BlockSpec((8, C), lambda i: (0, 0)),
        ],
        out_shape=[
            jax.ShapeDtypeStruct((N, C), f32),
            jax.ShapeDtypeStruct((8, C), f32),
        ],
    )(s1, st1, _row(norm1_g), _row(norm1_b), lin1_w.T, _row(lin1_b),
      lin2_w.T, _row(lin2_b))

    p, st3 = pl.pallas_call(
        _proj_body,
        grid=(N // BF,),
        in_specs=[
            pl.BlockSpec((BF, C), lambda i: (i, 0)),
            full((8, C)), full((1, C)), full((1, C)),
            full((C, C)), full((1, C)),
        ],
        out_specs=[
            pl.BlockSpec((BF, C), lambda i: (i, 0)),
            pl.BlockSpec((8, C), lambda i: (0, 0)),
        ],
        out_shape=[
            jax.ShapeDtypeStruct((N, C), f32),
            jax.ShapeDtypeStruct((8, C), f32),
        ],
    )(s2, st2, _row(norm2_g), _row(norm2_b), out_w.T, _row(out_b))

    out = pl.pallas_call(
        _final_body,
        grid=(N // BF,),
        in_specs=[
            pl.BlockSpec((BF, C), lambda i: (i, 0)),
            full((8, C)), full((1, C)), full((1, C)),
        ],
        out_specs=pl.BlockSpec((BF, C), lambda i: (i, 0)),
        out_shape=jax.ShapeDtypeStruct((N, C), f32),
    )(p, st3, _row(bn_out_g), _row(bn_out_b))

    return out


# speedup vs baseline: 1.4506x; 1.4506x over previous
import functools

import jax
import jax.numpy as jnp
from jax import lax
from jax.experimental import pallas as pl
from jax.experimental.pallas import tpu as pltpu
from jax.experimental.pallas import tpu_sc as plsc

N = 16384
C = 128
K = 48
FF = 256
H = 8
DH = C // H

SC_CORES = 2
SC_SUBCORES = 16
NW = SC_CORES * SC_SUBCORES
GCH = 128


def _pre_body(x_ref, idx_ref, ki_ref, wk3_ref, bk3_ref, wq3_ref, bq3_ref,
              wk_ref, bk_ref, wv_ref, bv_ref, wq_ref, bq_ref,
              kv_ref, q_ref, sidx_ref):
    idxf = idx_ref[...].astype(jnp.float32)
    cx = (idxf[:, 3:4] + 0.5) * 0.05
    cy = (idxf[:, 2:3] + 0.5) * 0.05 - 40.0
    cz = (idxf[:, 1:2] + 0.5) * 0.1 - 3.0
    cb = jnp.concatenate([cx, cy, cz], axis=1)
    x = x_ref[...]
    pos_k = jnp.maximum(
        jnp.dot(cb, wk3_ref[...], preferred_element_type=jnp.float32)
        + bk3_ref[...], 0.0)
    pos_q = jnp.maximum(
        jnp.dot(cb, wq3_ref[...], preferred_element_type=jnp.float32)
        + bq3_ref[...], 0.0)
    keysrc = x + pos_k
    kv_ref[:, :C] = jnp.dot(keysrc, wk_ref[...],
                            preferred_element_type=jnp.float32) + bk_ref[...]
    kv_ref[:, C:] = jnp.dot(keysrc, wv_ref[...],
                            preferred_element_type=jnp.float32) + bv_ref[...]
    q_ref[...] = (jnp.dot(x + pos_q, wq_ref[...],
                          preferred_element_type=jnp.float32)
                  + bq_ref[...]) * (1.0 / (DH ** 0.5))
    sidx_ref[...] = jnp.maximum(ki_ref[...], 0)


def _sc_gather_body(kv_hbm, sidx_hbm, out_hbm, idx_v,
                    rows0, rows1, rows2, sg0, sg1, sg2, sw0, sw1, sw2):
    wid = lax.axis_index("s") * SC_CORES + lax.axis_index("c")
    per_w = (N * K) // NW
    base = wid * per_w
    nch = per_w // GCH
    rows = (rows0, rows1, rows2)
    sg = (sg0, sg1, sg2)
    sw = (sw0, sw1, sw2)

    pltpu.sync_copy(sidx_hbm.at[pl.ds(base, per_w)], idx_v)

    def fire_gather(i, b):
        pltpu.async_copy(kv_hbm.at[idx_v.at[pl.ds(i * GCH, GCH)]],
                         rows[b], sg[b])

    def wait_gather(i, b):
        pltpu.make_async_copy(kv_hbm.at[idx_v.at[pl.ds(i * GCH, GCH)]],
                              rows[b], sg[b]).wait()

    def fire_wb(i, b):
        pltpu.async_copy(rows[b], out_hbm.at[pl.ds(base + i * GCH, GCH)],
                         sw[b])

    def wait_wb(i, b):
        pltpu.make_async_copy(rows[b],
                              out_hbm.at[pl.ds(base + i * GCH, GCH)],
                              sw[b]).wait()

    for b in range(3):
        fire_gather(b, b)

    def body(t, carry):
        i0 = 3 * t
        for b in range(3):
            i = i0 + b
            wait_gather(i, b)
            fire_wb(i, b)
            wait_wb(i, b)
            fire_gather(i + 3, b)
        return carry

    lax.fori_loop(0, nch // 3 - 1, body, 0)
    for b in range(3):
        i = nch - 3 + b
        wait_gather(i, b)
        fire_wb(i, b)
        wait_wb(i, b)


@functools.cache
def _make_sc_gather():
    return pl.kernel(
        _sc_gather_body,
        out_type=jax.ShapeDtypeStruct((N * K, 2 * C), jnp.float32),
        mesh=plsc.VectorSubcoreMesh(core_axis_name="c", subcore_axis_name="s",
                                    num_cores=SC_CORES,
                                    num_subcores=SC_SUBCORES),
        scratch_types=[
            pltpu.VMEM(((N * K) // NW,), jnp.int32),
            pltpu.VMEM((GCH, 2 * C), jnp.float32),
            pltpu.VMEM((GCH, 2 * C), jnp.float32),
            pltpu.VMEM((GCH, 2 * C), jnp.float32),
            pltpu.SemaphoreType.DMA,
            pltpu.SemaphoreType.DMA,
            pltpu.SemaphoreType.DMA,
            pltpu.SemaphoreType.DMA,
            pltpu.SemaphoreType.DMA,
            pltpu.SemaphoreType.DMA,
        ],
    )


def _sc_gather(kv_tab, sidx):
    return _make_sc_gather()(kv_tab, sidx)


def _attn_body(kv_ref, q_ref, x_ref, ki_ref, srep_ref, wo_ref, bo_ref,
               s1_ref, st_ref):
    b = q_ref.shape[0]
    kv = kv_ref[...]
    q = q_ref[...]
    kk = kv[:, :, :C]
    vv = kv[:, :, C:]
    prod = kk * q[:, None, :]
    logits = jnp.dot(prod.reshape(b * K, C), srep_ref[...],
                     preferred_element_type=jnp.float32,
                     precision=lax.Precision.HIGHEST).reshape(b, K, C)
    pen = jnp.where(ki_ref[...] < 0, -1e9, 0.0)
    logits = logits + lax.broadcast_in_dim(pen, (b, K, C), (0, 1))
    m = jnp.max(logits, axis=1, keepdims=True)
    e = jnp.exp(logits - m)
    s = jnp.sum(e, axis=1, keepdims=True)
    attn_full = e / s
    ctx = jnp.sum(attn_full * vv, axis=1)
    attend = jnp.dot(ctx, wo_ref[...],
                     preferred_element_type=jnp.float32) + bo_ref[...]
    s1 = x_ref[...] + attend
    s1_ref[...] = s1
    blk = jnp.concatenate(
        [jnp.sum(s1, axis=0, keepdims=True),
         jnp.sum(s1 * s1, axis=0, keepdims=True),
         jnp.zeros((6, C), jnp.float32)], axis=0)

    @pl.when(pl.program_id(0) == 0)
    def _():
        st_ref[...] = jnp.zeros_like(st_ref)

    st_ref[...] += blk


def _ffn_body(s1_ref, st1_ref, g1_ref, b1_ref, w1_ref, bb1_ref,
              w2_ref, bb2_ref, s2_ref, st_ref):
    mean = st1_ref[0:1, :] * (1.0 / N)
    var = st1_ref[1:2, :] * (1.0 / N) - mean * mean
    vf = ((s1_ref[...] - mean) * lax.rsqrt(var + 1e-5) * g1_ref[...]
          + b1_ref[...])
    a = jnp.maximum(jnp.dot(vf, w1_ref[...],
                            preferred_element_type=jnp.float32)
                    + bb1_ref[...], 0.0)
    act = jnp.dot(a, w2_ref[...],
                  preferred_element_type=jnp.float32) + bb2_ref[...]
    s2 = vf + act
    s2_ref[...] = s2
    blk = jnp.concatenate(
        [jnp.sum(s2, axis=0, keepdims=True),
         jnp.sum(s2 * s2, axis=0, keepdims=True),
         jnp.zeros((6, C), jnp.float32)], axis=0)

    @pl.when(pl.program_id(0) == 0)
    def _():
        st_ref[...] = jnp.zeros_like(st_ref)

    st_ref[...] += blk


def _proj_body(s2_ref, st2_ref, g2_ref, b2_ref, wo_ref, bo_ref,
               p_ref, st_ref):
    mean = st2_ref[0:1, :] * (1.0 / N)
    var = st2_ref[1:2, :] * (1.0 / N) - mean * mean
    vf = ((s2_ref[...] - mean) * lax.rsqrt(var + 1e-5) * g2_ref[...]
          + b2_ref[...])
    p = jnp.dot(vf, wo_ref[...],
                preferred_element_type=jnp.float32) + bo_ref[...]
    p_ref[...] = p
    blk = jnp.concatenate(
        [jnp.sum(p, axis=0, keepdims=True),
         jnp.sum(p * p, axis=0, keepdims=True),
         jnp.zeros((6, C), jnp.float32)], axis=0)

    @pl.when(pl.program_id(0) == 0)
    def _():
        st_ref[...] = jnp.zeros_like(st_ref)

    st_ref[...] += blk


def _final_body(p_ref, st3_ref, g3_ref, b3_ref, out_ref):
    mean = st3_ref[0:1, :] * (1.0 / N)
    var = st3_ref[1:2, :] * (1.0 / N) - mean * mean
    out_ref[...] = jnp.maximum(
        (p_ref[...] - mean) * lax.rsqrt(var + 1e-5) * g3_ref[...]
        + b3_ref[...], 0.0)


def _row(v):
    return v.reshape(1, -1)


def kernel(x, indices, key_indices, in_proj_w, in_proj_b, out_proj_w,
           out_proj_b, q_pos_w, q_pos_b, k_pos_w, k_pos_b, norm1_g, norm1_b,
           norm2_g, norm2_b, lin1_w, lin1_b, lin2_w, lin2_b, out_w, out_b,
           bn_out_g, bn_out_b):
    f32 = jnp.float32
    wq = in_proj_w[:C].T
    wk = in_proj_w[C:2 * C].T
    wv = in_proj_w[2 * C:].T
    bq, bk, bv = in_proj_b[:C], in_proj_b[C:2 * C], in_proj_b[2 * C:]

    BP = 1024
    full = lambda shape: pl.BlockSpec(shape, lambda i: (0, 0))
    kv_tab, q_tab, sidx = pl.pallas_call(
        _pre_body,
        grid=(N // BP,),
        in_specs=[
            pl.BlockSpec((BP, C), lambda i: (i, 0)),
            pl.BlockSpec((BP, 4), lambda i: (i, 0)),
            pl.BlockSpec((BP, K), lambda i: (i, 0)),
            full((3, C)), full((1, C)), full((3, C)), full((1, C)),
            full((C, C)), full((1, C)), full((C, C)), full((1, C)),
            full((C, C)), full((1, C)),
        ],
        out_specs=[
            pl.BlockSpec((BP, 2 * C), lambda i: (i, 0)),
            pl.BlockSpec((BP, C), lambda i: (i, 0)),
            pl.BlockSpec((BP, K), lambda i: (i, 0)),
        ],
        out_shape=[
            jax.ShapeDtypeStruct((N, 2 * C), f32),
            jax.ShapeDtypeStruct((N, C), f32),
            jax.ShapeDtypeStruct((N, K), jnp.int32),
        ],
    )(x, indices, key_indices, k_pos_w.T, _row(k_pos_b), q_pos_w.T,
      _row(q_pos_b), wk, _row(bk), wv, _row(bv), wq, _row(bq))

    kv_gathered = _sc_gather(kv_tab, sidx.reshape(N * K))
    kv_gathered = kv_gathered.reshape(N, K, 2 * C)

    BA = 128
    lane = jnp.arange(C, dtype=jnp.int32)
    srep = (lane[:, None] // DH == lane[None, :] // DH).astype(f32)
    s1, st1 = pl.pallas_call(
        _attn_body,
        grid=(N // BA,),
        in_specs=[
            pl.BlockSpec((BA, K, 2 * C), lambda i: (i, 0, 0)),
            pl.BlockSpec((BA, C), lambda i: (i, 0)),
            pl.BlockSpec((BA, C), lambda i: (i, 0)),
            pl.BlockSpec((BA, K), lambda i: (i, 0)),
            full((C, C)), full((C, C)), full((1, C)),
        ],
        out_specs=[
            pl.BlockSpec((BA, C), lambda i: (i, 0)),
            pl.BlockSpec((8, C), lambda i: (0, 0)),
        ],
        out_shape=[
            jax.ShapeDtypeStruct((N, C), f32),
            jax.ShapeDtypeStruct((8, C), f32),
        ],
    )(kv_gathered, q_tab, x, key_indices, srep, out_proj_w.T,
      _row(out_proj_b))

    BF = 2048
    s2, st2 = pl.pallas_call(
        _ffn_body,
        grid=(N // BF,),
        in_specs=[
            pl.BlockSpec((BF, C), lambda i: (i, 0)),
            full((8, C)), full((1, C)), full((1, C)),
            full((C, FF)), full((1, FF)), full((FF, C)), full((1, C)),
        ],
        out_specs=[
            pl.BlockSpec((BF, C), lambda i: (i, 0)),
            pl.BlockSpec((8, C), lambda i: (0, 0)),
        ],
        out_shape=[
            jax.ShapeDtypeStruct((N, C), f32),
            jax.ShapeDtypeStruct((8, C), f32),
        ],
    )(s1, st1, _row(norm1_g), _row(norm1_b), lin1_w.T, _row(lin1_b),
      lin2_w.T, _row(lin2_b))

    p, st3 = pl.pallas_call(
        _proj_body,
        grid=(N // BF,),
        in_specs=[
            pl.BlockSpec((BF, C), lambda i: (i, 0)),
            full((8, C)), full((1, C)), full((1, C)),
            full((C, C)), full((1, C)),
        ],
        out_specs=[
            pl.BlockSpec((BF, C), lambda i: (i, 0)),
            pl.BlockSpec((8, C), lambda i: (0, 0)),
        ],
        out_shape=[
            jax.ShapeDtypeStruct((N, C), f32),
            jax.ShapeDtypeStruct((8, C), f32),
        ],
    )(s2, st2, _row(norm2_g), _row(norm2_b), out_w.T, _row(out_b))

    out = pl.pallas_call(
        _final_body,
        grid=(N // BF,),
        in_specs=[
            pl.BlockSpec((BF, C), lambda i: (i, 0)),
            full((8, C)), full((1, C)), full((1, C)),
        ],
        out_specs=pl.BlockSpec((BF, C), lambda i: (i, 0)),
        out_shape=jax.ShapeDtypeStruct((N, C), f32),
    )(p, st3, _row(bn_out_g), _row(bn_out_b))

    return out
